# baseline (speedup 1.0000x reference)
import functools

import jax
import jax.numpy as jnp
from jax import lax
from jax.experimental import pallas as pl
from jax.experimental.pallas import tpu as pltpu
from jax.experimental.pallas import tpu_sc as plsc

D_MODEL = 2048
SEQ_LEN = 2048

NUM_CORES = 2
NUM_SUBCORES = 16
NUM_WORKERS = NUM_CORES * NUM_SUBCORES

CHUNK = 16
VECS_PER_ROW = D_MODEL // 16


def _sc_embed(seq_flat, table, pe):
    num_tokens = seq_flat.shape[0]
    per_worker = num_tokens // NUM_WORKERS
    num_chunks = per_worker // CHUNK
    mesh = plsc.VectorSubcoreMesh(core_axis_name="c", subcore_axis_name="s")

    @functools.partial(
        pl.kernel,
        out_type=jax.ShapeDtypeStruct((num_tokens, D_MODEL), jnp.float32),
        mesh=mesh,
        scratch_types=[
            pltpu.VMEM((per_worker,), jnp.int32),
            pltpu.VMEM((2, CHUNK, D_MODEL), jnp.float32),
            pltpu.VMEM((CHUNK, D_MODEL), jnp.float32),
            pltpu.SemaphoreType.DMA,
            pltpu.SemaphoreType.DMA,
            pltpu.SemaphoreType.DMA,
            pltpu.SemaphoreType.DMA,
        ],
    )
    def k(seq_hbm, table_hbm, pe_hbm, out_hbm, idx_v, rows, pe_v, sg0, sg1, sw0, sw1):
        sg = [sg0, sg1]
        sw = [sw0, sw1]
        wid = lax.axis_index("s") * NUM_CORES + lax.axis_index("c")
        base = wid * per_worker
        pos0 = lax.rem(base, SEQ_LEN)
        pltpu.sync_copy(seq_hbm.at[pl.ds(base, per_worker)], idx_v)

        def gather_start(c):
            s = c % 2
            return pltpu.async_copy(
                table_hbm.at[idx_v.at[pl.ds(c * CHUNK, CHUNK)]], rows.at[s], sg[s]
            )

        wb = [None, None]
        g = [None] * (num_chunks + 1)
        g[0] = gather_start(0)
        for c in range(num_chunks):
            s = c % 2
            s2 = (c + 1) % 2
            if c + 1 < num_chunks:
                if wb[s2] is not None:
                    wb[s2].wait()
                    wb[s2] = None
                g[c + 1] = gather_start(c + 1)
            pltpu.sync_copy(pe_hbm.at[pl.ds(pos0 + c * CHUNK, CHUNK)], pe_v)
            g[c].wait()

            @plsc.parallel_loop(0, CHUNK * VECS_PER_ROW // 8, unroll=4)
            def add_group(i):
                j = lax.shift_right_logical(i, 4)
                colbase = pl.multiple_of(
                    lax.shift_left(lax.bitwise_and(i, 15), 7), 16
                )
                for u in range(8):
                    plsc.addupdate(
                        rows.at[s, j, pl.ds(colbase + u * 16, 16)],
                        pe_v[j, pl.ds(colbase + u * 16, 16)],
                    )

            wb[s] = pltpu.async_copy(
                rows.at[s], out_hbm.at[pl.ds(base + c * CHUNK, CHUNK)], sw[s]
            )
        for s in range(2):
            if wb[s] is not None:
                wb[s].wait()

    return k(seq_flat, table, pe)


def kernel(seq, table, pe):
    batch, seq_len = seq.shape
    seq_flat = seq.reshape(-1).astype(jnp.int32)
    out = _sc_embed(seq_flat, table, pe)
    return out.reshape(batch, seq_len, D_MODEL)


# speedup vs baseline: 1.3164x; 1.3164x over previous
import functools

import jax
import jax.numpy as jnp
from jax import lax
from jax.experimental import pallas as pl
from jax.experimental.pallas import tpu as pltpu
from jax.experimental.pallas import tpu_sc as plsc

D_MODEL = 2048
SEQ_LEN = 2048

NUM_CORES = 2
NUM_SUBCORES = 16
NUM_WORKERS = NUM_CORES * NUM_SUBCORES

CHUNK = 8
VECS_PER_ROW = D_MODEL // 16


def _sc_embed(seq_flat, table, pe):
    num_tokens = seq_flat.shape[0]
    per_worker = num_tokens // NUM_WORKERS
    num_chunks = per_worker // CHUNK
    mesh = plsc.VectorSubcoreMesh(core_axis_name="c", subcore_axis_name="s")

    @functools.partial(
        pl.kernel,
        out_type=jax.ShapeDtypeStruct((num_tokens, D_MODEL), jnp.float32),
        mesh=mesh,
        scratch_types=[
            pltpu.VMEM((per_worker,), jnp.int32),
            pltpu.VMEM((2, CHUNK, D_MODEL), jnp.float32),
            pltpu.VMEM((2, CHUNK, D_MODEL), jnp.float32),
            pltpu.SemaphoreType.DMA,
            pltpu.SemaphoreType.DMA,
            pltpu.SemaphoreType.DMA,
            pltpu.SemaphoreType.DMA,
            pltpu.SemaphoreType.DMA,
            pltpu.SemaphoreType.DMA,
        ],
    )
    def k(seq_hbm, table_hbm, pe_hbm, out_hbm, idx_v, rows, pe_v,
          sg0, sg1, sp0, sp1, sw0, sw1):
        sg = [sg0, sg1]
        sp = [sp0, sp1]
        sw = [sw0, sw1]
        wid = lax.axis_index("s") * NUM_CORES + lax.axis_index("c")
        base = wid * per_worker
        pos0 = lax.rem(base, SEQ_LEN)
        pltpu.sync_copy(seq_hbm.at[pl.ds(base, per_worker)], idx_v)

        def gather_start(c):
            s = c % 2
            return pltpu.async_copy(
                table_hbm.at[idx_v.at[pl.ds(c * CHUNK, CHUNK)]], rows.at[s], sg[s]
            )

        def pe_start(c):
            s = c % 2
            return pltpu.async_copy(
                pe_hbm.at[pl.ds(pos0 + c * CHUNK, CHUNK)], pe_v.at[s], sp[s]
            )

        wb = [None, None]
        g = [None] * (num_chunks + 1)
        p = [None] * (num_chunks + 1)
        g[0] = gather_start(0)
        p[0] = pe_start(0)
        for c in range(num_chunks):
            s = c % 2
            s2 = (c + 1) % 2
            if c + 1 < num_chunks:
                if wb[s2] is not None:
                    wb[s2].wait()
                    wb[s2] = None
                g[c + 1] = gather_start(c + 1)
                p[c + 1] = pe_start(c + 1)
            g[c].wait()
            p[c].wait()

            @plsc.parallel_loop(0, CHUNK * VECS_PER_ROW // 8, unroll=4)
            def add_group(i):
                j = lax.shift_right_logical(i, 4)
                colbase = pl.multiple_of(
                    lax.shift_left(lax.bitwise_and(i, 15), 7), 16
                )
                for u in range(8):
                    plsc.addupdate(
                        rows.at[s, j, pl.ds(colbase + u * 16, 16)],
                        pe_v[s, j, pl.ds(colbase + u * 16, 16)],
                    )

            wb[s] = pltpu.async_copy(
                rows.at[s], out_hbm.at[pl.ds(base + c * CHUNK, CHUNK)], sw[s]
            )
        for s in range(2):
            if wb[s] is not None:
                wb[s].wait()

    return k(seq_flat, table, pe)


def kernel(seq, table, pe):
    batch, seq_len = seq.shape
    seq_flat = seq.reshape(-1).astype(jnp.int32)
    out = _sc_embed(seq_flat, table, pe)
    return out.reshape(batch, seq_len, D_MODEL)


# speedup vs baseline: 1.3490x; 1.0247x over previous
import functools

import jax
import jax.numpy as jnp
from jax import lax
from jax.experimental import pallas as pl
from jax.experimental.pallas import tpu as pltpu
from jax.experimental.pallas import tpu_sc as plsc

D_MODEL = 2048
SEQ_LEN = 2048

NUM_CORES = 2
NUM_SUBCORES = 16
NUM_WORKERS = NUM_CORES * NUM_SUBCORES

CHUNK = 8
VECS_PER_ROW = D_MODEL // 16


def _sc_embed(seq_flat, table, pe):
    num_tokens = seq_flat.shape[0]
    per_worker = num_tokens // NUM_WORKERS
    num_chunks = per_worker // CHUNK
    mesh = plsc.VectorSubcoreMesh(core_axis_name="c", subcore_axis_name="s")

    @functools.partial(
        pl.kernel,
        out_type=jax.ShapeDtypeStruct((num_tokens, D_MODEL), jnp.float32),
        mesh=mesh,
        scratch_types=[
            pltpu.VMEM((per_worker,), jnp.int32),
            pltpu.VMEM((4, CHUNK, D_MODEL), jnp.float32),
            pltpu.VMEM((2, CHUNK, D_MODEL), jnp.float32),
            pltpu.SemaphoreType.DMA,
            pltpu.SemaphoreType.DMA,
            pltpu.SemaphoreType.DMA,
            pltpu.SemaphoreType.DMA,
            pltpu.SemaphoreType.DMA,
            pltpu.SemaphoreType.DMA,
            pltpu.SemaphoreType.DMA,
            pltpu.SemaphoreType.DMA,
            pltpu.SemaphoreType.DMA,
            pltpu.SemaphoreType.DMA,
        ],
    )
    def k(seq_hbm, table_hbm, pe_hbm, out_hbm, idx_v, rows, pe_v,
          sg0, sg1, sg2, sg3, sp0, sp1, sw0, sw1, sw2, sw3):
        sg = [sg0, sg1, sg2, sg3]
        sp = [sp0, sp1]
        sw = [sw0, sw1, sw2, sw3]
        wid = lax.axis_index("s") * NUM_CORES + lax.axis_index("c")
        base = wid * per_worker
        pos0 = lax.rem(base, SEQ_LEN)
        pltpu.sync_copy(seq_hbm.at[pl.ds(base, per_worker)], idx_v)

        def gather_start(c):
            s = c % 4
            return pltpu.async_copy(
                table_hbm.at[idx_v.at[pl.ds(c * CHUNK, CHUNK)]], rows.at[s], sg[s]
            )

        def pe_start(c):
            s = c % 2
            return pltpu.async_copy(
                pe_hbm.at[pl.ds(pos0 + c * CHUNK, CHUNK)], pe_v.at[s], sp[s]
            )

        wb = [None, None, None, None]
        g = [None] * (num_chunks + 4)
        p = [None] * (num_chunks + 2)
        for d in range(3):
            g[d] = gather_start(d)
        p[0] = pe_start(0)
        for c in range(num_chunks):
            s = c % 4
            sp_slot = c % 2
            tgt = c + 3
            if tgt < num_chunks:
                st = tgt % 4
                if wb[st] is not None:
                    wb[st].wait()
                    wb[st] = None
                g[tgt] = gather_start(tgt)
            if c + 1 < num_chunks:
                p[c + 1] = pe_start(c + 1)
            g[c].wait()
            p[c].wait()

            @plsc.parallel_loop(0, CHUNK * VECS_PER_ROW // 8, unroll=4)
            def add_group(i):
                j = lax.shift_right_logical(i, 4)
                colbase = pl.multiple_of(
                    lax.shift_left(lax.bitwise_and(i, 15), 7), 16
                )
                for u in range(8):
                    plsc.addupdate(
                        rows.at[s, j, pl.ds(colbase + u * 16, 16)],
                        pe_v[sp_slot, j, pl.ds(colbase + u * 16, 16)],
                    )

            wb[s] = pltpu.async_copy(
                rows.at[s], out_hbm.at[pl.ds(base + c * CHUNK, CHUNK)], sw[s]
            )
        for s in range(4):
            if wb[s] is not None:
                wb[s].wait()

    return k(seq_flat, table, pe)


def kernel(seq, table, pe):
    batch, seq_len = seq.shape
    seq_flat = seq.reshape(-1).astype(jnp.int32)
    out = _sc_embed(seq_flat, table, pe)
    return out.reshape(batch, seq_len, D_MODEL)


# speedup vs baseline: 1.5099x; 1.1193x over previous
import functools

import jax
import jax.numpy as jnp
from jax import lax
from jax.experimental import pallas as pl
from jax.experimental.pallas import tpu as pltpu
from jax.experimental.pallas import tpu_sc as plsc

D_MODEL = 2048
SEQ_LEN = 2048

NUM_CORES = 2
NUM_SUBCORES = 16
NUM_WORKERS = NUM_CORES * NUM_SUBCORES

CHUNK = 8
VECS_PER_ROW = D_MODEL // 16


def _sc_embed(seq_posmajor, table, pe, out_row_idx, batch):
    num_tokens = seq_posmajor.shape[0]
    per_worker = num_tokens // NUM_WORKERS
    num_chunks = per_worker // CHUNK
    pos_per_chunk = CHUNK // batch
    mesh = plsc.VectorSubcoreMesh(core_axis_name="c", subcore_axis_name="s")

    @functools.partial(
        pl.kernel,
        out_type=jax.ShapeDtypeStruct((num_tokens, D_MODEL), jnp.float32),
        mesh=mesh,
        scratch_types=[
            pltpu.VMEM((per_worker,), jnp.int32),
            pltpu.VMEM((num_chunks, CHUNK), jnp.int32),
            pltpu.VMEM((4, CHUNK, D_MODEL), jnp.float32),
            pltpu.VMEM((2, 8, D_MODEL), jnp.float32),
            pltpu.SemaphoreType.DMA,
            pltpu.SemaphoreType.DMA,
            pltpu.SemaphoreType.DMA,
            pltpu.SemaphoreType.DMA,
            pltpu.SemaphoreType.DMA,
            pltpu.SemaphoreType.DMA,
            pltpu.SemaphoreType.DMA,
            pltpu.SemaphoreType.DMA,
            pltpu.SemaphoreType.DMA,
            pltpu.SemaphoreType.DMA,
        ],
    )
    def k(seq_hbm, table_hbm, pe_hbm, oidx_hbm, out_hbm, idx_v, oidx_v, rows,
          pe_v, sg0, sg1, sg2, sg3, sp0, sp1, sw0, sw1, sw2, sw3):
        sg = [sg0, sg1, sg2, sg3]
        sp = [sp0, sp1]
        sw = [sw0, sw1, sw2, sw3]
        wid = lax.axis_index("s") * NUM_CORES + lax.axis_index("c")
        base = wid * per_worker
        pos0 = base // batch
        pltpu.sync_copy(seq_hbm.at[pl.ds(base, per_worker)], idx_v)
        pltpu.sync_copy(oidx_hbm.at[wid], oidx_v)

        def gather_start(c):
            s = c % 4
            return pltpu.async_copy(
                table_hbm.at[idx_v.at[pl.ds(c * CHUNK, CHUNK)]], rows.at[s], sg[s]
            )

        chunks_per_pe = 8 // pos_per_chunk
        num_pe_chunks = num_chunks // chunks_per_pe

        def pe_start(cp):
            s = cp % 2
            return pltpu.async_copy(
                pe_hbm.at[pl.ds(pl.multiple_of(pos0 + cp * 8, 8), 8)],
                pe_v.at[s],
                sp[s],
            )

        wb = [None, None, None, None]
        g = [None] * (num_chunks + 4)
        p = [None] * (num_pe_chunks + 1)
        for d in range(3):
            g[d] = gather_start(d)
        p[0] = pe_start(0)
        for c in range(num_chunks):
            s = c % 4
            cp = c // chunks_per_pe
            sp_slot = cp % 2
            row0 = (c % chunks_per_pe) * pos_per_chunk
            tgt = c + 3
            if tgt < num_chunks:
                st = tgt % 4
                if wb[st] is not None:
                    wb[st].wait()
                    wb[st] = None
                g[tgt] = gather_start(tgt)
            if c % chunks_per_pe == 0:
                if cp + 1 < num_pe_chunks:
                    p[cp + 1] = pe_start(cp + 1)
                p[cp].wait()
            g[c].wait()

            @plsc.parallel_loop(0, CHUNK * VECS_PER_ROW // 8, unroll=4)
            def add_group(i):
                j = lax.shift_right_logical(i, 4)
                jp = row0 + lax.shift_right_logical(i, 6)
                colbase = pl.multiple_of(
                    lax.shift_left(lax.bitwise_and(i, 15), 7), 16
                )
                for u in range(8):
                    plsc.addupdate(
                        rows.at[s, j, pl.ds(colbase + u * 16, 16)],
                        pe_v[sp_slot, jp, pl.ds(colbase + u * 16, 16)],
                    )

            wb[s] = pltpu.async_copy(
                rows.at[s], out_hbm.at[oidx_v.at[c]], sw[s]
            )
        for s in range(4):
            if wb[s] is not None:
                wb[s].wait()

    return k(seq_posmajor, table, pe, out_row_idx)


def kernel(seq, table, pe):
    batch, seq_len = seq.shape
    num_tokens = batch * seq_len
    per_worker = num_tokens // NUM_WORKERS
    num_chunks = per_worker // CHUNK
    seq_pm = jnp.swapaxes(seq, 0, 1).reshape(-1).astype(jnp.int32)
    t = jnp.arange(num_tokens, dtype=jnp.int32)
    out_row_idx = (t % batch) * seq_len + t // batch
    out_row_idx = out_row_idx.reshape(NUM_WORKERS, num_chunks, CHUNK)
    out = _sc_embed(seq_pm, table, pe, out_row_idx, batch)
    return out.reshape(batch, seq_len, D_MODEL)


# speedup vs baseline: 1.7123x; 1.1340x over previous
import functools

import jax
import jax.numpy as jnp
from jax import lax
from jax.experimental import pallas as pl
from jax.experimental.pallas import tpu as pltpu
from jax.experimental.pallas import tpu_sc as plsc

D_MODEL = 2048
SEQ_LEN = 2048

NUM_CORES = 2
NUM_SUBCORES = 16
NUM_WORKERS = NUM_CORES * NUM_SUBCORES

CHUNK = 8
VECS_PER_ROW = D_MODEL // 16


def _sc_embed(seq_posmajor, table, pe, out_row_idx, batch):
    num_tokens = seq_posmajor.shape[0]
    per_worker = num_tokens // NUM_WORKERS
    num_chunks = per_worker // CHUNK
    pos_per_chunk = CHUNK // batch
    mesh = plsc.VectorSubcoreMesh(core_axis_name="c", subcore_axis_name="s")

    @functools.partial(
        pl.kernel,
        out_type=jax.ShapeDtypeStruct((num_tokens, D_MODEL), jnp.float32),
        mesh=mesh,
        scratch_types=[
            pltpu.VMEM((per_worker,), jnp.int32),
            pltpu.VMEM((num_chunks, CHUNK), jnp.int32),
            pltpu.VMEM((4, CHUNK, D_MODEL), jnp.float32),
            pltpu.VMEM((2, 8, D_MODEL), jnp.float32),
            pltpu.SemaphoreType.DMA,
            pltpu.SemaphoreType.DMA,
            pltpu.SemaphoreType.DMA,
            pltpu.SemaphoreType.DMA,
            pltpu.SemaphoreType.DMA,
            pltpu.SemaphoreType.DMA,
            pltpu.SemaphoreType.DMA,
            pltpu.SemaphoreType.DMA,
            pltpu.SemaphoreType.DMA,
            pltpu.SemaphoreType.DMA,
        ],
    )
    def k(seq_hbm, table_hbm, pe_hbm, oidx_hbm, out_hbm, idx_v, oidx_v, rows,
          pe_v, sg0, sg1, sg2, sg3, sp0, sp1, sw0, sw1, sw2, sw3):
        sg = [sg0, sg1, sg2, sg3]
        sp = [sp0, sp1]
        sw = [sw0, sw1, sw2, sw3]
        wid = lax.axis_index("s") * NUM_CORES + lax.axis_index("c")
        base = wid * per_worker
        pos0 = base // batch
        pltpu.sync_copy(seq_hbm.at[pl.ds(base, per_worker)], idx_v)
        pltpu.sync_copy(oidx_hbm.at[wid], oidx_v)

        def gather_start(c):
            s = c % 4
            return pltpu.async_copy(
                table_hbm.at[idx_v.at[pl.ds(c * CHUNK, CHUNK)]], rows.at[s], sg[s]
            )

        chunks_per_pe = 8 // pos_per_chunk
        num_pe_chunks = num_chunks // chunks_per_pe

        def pe_start(cp):
            s = cp % 2
            return pltpu.async_copy(
                pe_hbm.at[pl.ds(pl.multiple_of(pos0 + cp * 8, 8), 8)],
                pe_v.at[s],
                sp[s],
            )

        wb = [None, None, None, None]
        g = [None] * (num_chunks + 4)
        p = [None] * (num_pe_chunks + 1)
        for d in range(3):
            g[d] = gather_start(d)
        p[0] = pe_start(0)
        for c in range(num_chunks):
            s = c % 4
            cp = c // chunks_per_pe
            sp_slot = cp % 2
            row0 = (c % chunks_per_pe) * pos_per_chunk
            tgt = c + 3
            if tgt < num_chunks:
                st = tgt % 4
                if wb[st] is not None:
                    wb[st].wait()
                    wb[st] = None
                g[tgt] = gather_start(tgt)
            if c % chunks_per_pe == 0:
                if cp + 1 < num_pe_chunks:
                    p[cp + 1] = pe_start(cp + 1)
                p[cp].wait()
            g[c].wait()

            @plsc.parallel_loop(0, pos_per_chunk * VECS_PER_ROW // 2, unroll=4)
            def add_group(i):
                pidx = lax.shift_right_logical(i, 6)
                jp = row0 + pidx
                colbase = pl.multiple_of(
                    lax.shift_left(lax.bitwise_and(i, 63), 5), 16
                )
                for u in range(2):
                    col = colbase + u * 16
                    pv = pe_v[sp_slot, jp, pl.ds(col, 16)]
                    for b in range(batch):
                        plsc.addupdate(
                            rows.at[s, pidx * batch + b, pl.ds(col, 16)], pv
                        )

            wb[s] = pltpu.async_copy(
                rows.at[s], out_hbm.at[oidx_v.at[c]], sw[s]
            )
        for s in range(4):
            if wb[s] is not None:
                wb[s].wait()

    return k(seq_posmajor, table, pe, out_row_idx)


def kernel(seq, table, pe):
    batch, seq_len = seq.shape
    num_tokens = batch * seq_len
    per_worker = num_tokens // NUM_WORKERS
    num_chunks = per_worker // CHUNK
    seq_pm = jnp.swapaxes(seq, 0, 1).reshape(-1).astype(jnp.int32)
    t = jnp.arange(num_tokens, dtype=jnp.int32)
    out_row_idx = (t % batch) * seq_len + t // batch
    out_row_idx = out_row_idx.reshape(NUM_WORKERS, num_chunks, CHUNK)
    out = _sc_embed(seq_pm, table, pe, out_row_idx, batch)
    return out.reshape(batch, seq_len, D_MODEL)
